# baseline (speedup 1.0000x reference)
import functools
import numpy as np
import jax
import jax.numpy as jnp
from jax import lax
from jax.experimental import pallas as pl
from jax.experimental.pallas import tpu as pltpu
from jax.experimental.pallas import tpu_sc as plsc

_FR = [
    ((0.0, 0.33), (0.0, 0.33), (0.0, 0.33), 0.0),
    ((0.0, 0.33), (0.0, 0.33), (0.67, 1.0), 0.333),
    ((0.0, 0.33), (0.33, 0.67), (0.33, 0.67), 0.333),
    ((0.0, 0.33), (0.67, 1.0), (0.0, 0.33), 0.333),
    ((0.0, 0.33), (0.67, 1.0), (0.67, 1.0), 0.667),
    ((0.33, 0.67), (0.0, 0.33), (0.0, 0.33), 0.333),
    ((0.33, 0.67), (0.0, 0.33), (0.67, 1.0), 0.667),
    ((0.33, 0.67), (0.33, 0.67), (0.33, 0.67), 0.5),
    ((0.33, 0.67), (0.67, 1.0), (0.0, 0.33), 0.667),
    ((0.33, 0.67), (0.67, 1.0), (0.67, 1.0), 1.0),
    ((0.67, 1.0), (0.0, 0.33), (0.0, 0.33), 0.667),
    ((0.67, 1.0), (0.0, 0.33), (0.67, 1.0), 1.0),
    ((0.67, 1.0), (0.33, 0.67), (0.33, 0.67), 1.0),
    ((0.67, 1.0), (0.67, 1.0), (0.0, 0.33), 1.167),
    ((0.67, 1.0), (0.67, 1.0), (0.67, 1.0), 1.5),
]
_LO = np.array([[r[0][0], r[1][0], r[2][0]] for r in _FR], dtype=np.float32)
_HI = np.array([[r[0][1], r[1][1], r[2][1]] for r in _FR], dtype=np.float32)
_CONS = np.array([r[3] for r in _FR], dtype=np.float32)
_CEN = (_LO + _HI) / 2.0


def _ent_body(x_ref, x1_ref, out_ent_ref, par_ref, ent_s, sum_s, img_s,
              *, B, S, D, BS, NBLK):
    b = pl.program_id(0)
    j = pl.program_id(1)

    xb = x_ref[0]
    m = jnp.max(xb, axis=1, keepdims=True)
    t = xb - m
    e = jnp.exp(t)
    Z = jnp.sum(e, axis=1, keepdims=True)
    sxe = jnp.sum(e * t, axis=1, keepdims=True)
    ent = jnp.log(Z) - sxe / Z

    ent_s[pl.ds(b, 1), pl.ds(j * BS, BS)] = ent.reshape(1, BS)

    part = jnp.sum(xb, axis=0, keepdims=True)

    @pl.when(j == 0)
    def _():
        sum_s[pl.ds(b, 1), :] = part
        img_s[pl.ds(b, 1), :] = xb[0:1, :]

    @pl.when(j != 0)
    def _():
        sum_s[pl.ds(b, 1), :] = sum_s[pl.ds(b, 1), :] + part

    @pl.when((b == B - 1) & (j == NBLK - 1))
    def _():
        ent_all = ent_s[:, :]
        img = img_s[:, :]
        tot = sum_s[:, :]
        out_ent_ref[:, :] = ent_all

        cols = jax.lax.broadcasted_iota(jnp.int32, (B, S), 1)
        ent_txt = jnp.where(cols >= 1, ent_all, -jnp.inf)
        rtx = 1.0 / jnp.maximum(jnp.max(ent_txt), 1e-6)

        bmean = (tot - img) / float(S - 1)
        dot = jnp.sum(img * bmean, axis=1, keepdims=True)
        na = jnp.sqrt(jnp.sum(img * img, axis=1, keepdims=True))
        nb = jnp.sqrt(jnp.sum(bmean * bmean, axis=1, keepdims=True))
        x3 = dot / jnp.clip(na * nb, 1e-8, None)

        lanes = jax.lax.broadcasted_iota(jnp.int32, (B, 48), 1)
        x1 = x1_ref[:, :]
        par_ref[:, :] = jnp.where(lanes < 16, x1,
                                  jnp.where(lanes < 32, x3, rtx))


def _nsqrt(d):
    bits = plsc.bitcast(d, jnp.int32)
    s = plsc.bitcast((bits >> 1) + 0x1FBD1DF5, jnp.float32)
    for _ in range(3):
        s = 0.5 * (s + d / jnp.maximum(s, 1e-30))
    return jnp.where(d > 0.0, s, 0.0)


def _route_body(ent_hbm, params_hbm, w_hbm, ent_v, par_v, w_v,
                *, TPW, S, WPB):
    c = lax.axis_index("c")
    s = lax.axis_index("s")
    wid = c * 16 + s
    base = wid * TPW
    pltpu.sync_copy(ent_hbm.at[pl.ds(base, TPW)], ent_v)
    pltpu.sync_copy(params_hbm.at[wid // WPB], par_v)
    x1 = par_v[pl.ds(0, 16)]
    x3 = par_v[pl.ds(16, 16)]
    rtx = par_v[pl.ds(32, 16)]
    lane = lax.iota(jnp.int32, 16)

    bb = []
    kr = []
    for r in range(15):
        lo0, _, lo2 = (float(v) for v in _LO[r])
        hi0, _, hi2 = (float(v) for v in _HI[r])
        ce0, _, ce2 = (float(v) for v in _CEN[r])
        bb.append((x1 >= lo0) & (x1 < hi0) & (x3 >= lo2) & (x3 < hi2))
        dx = x1 - ce0
        dz = x3 - ce2
        kr.append(dx * dx + dz * dz)

    for j in range(TPW // 16):
        e = ent_v[pl.ds(j * 16, 16)]
        x2 = e * rtx
        any_m = jnp.zeros((16,), jnp.bool_)
        mval = jnp.zeros((16,), jnp.float32)
        d1 = jnp.full((16,), jnp.inf, jnp.float32)
        d2 = jnp.full((16,), jnp.inf, jnp.float32)
        c1 = jnp.zeros((16,), jnp.float32)
        c2 = jnp.zeros((16,), jnp.float32)
        for r in range(15):
            lo1 = float(_LO[r][1])
            hi1 = float(_HI[r][1])
            ce1 = float(_CEN[r][1])
            cons = float(_CONS[r])
            m_r = bb[r] & (x2 >= lo1) & (x2 < hi1)
            any_m = any_m | m_r
            mval = mval + jnp.where(m_r, cons, 0.0)
            dy = x2 - ce1
            dq = dy * dy + kr[r]
            lt1 = dq < d1
            lt2 = (dq < d2) & (~lt1)
            d2n = jnp.where(lt1, d1, jnp.where(lt2, dq, d2))
            c2n = jnp.where(lt1, c1, jnp.where(lt2, cons, c2))
            d1 = jnp.where(lt1, dq, d1)
            c1 = jnp.where(lt1, cons, c1)
            d2 = d2n
            c2 = c2n
        s1 = _nsqrt(d1)
        s2 = _nsqrt(d2)
        dsum = s1 + s2
        lam = jnp.where(dsum != 0.0,
                        s1 / jnp.where(dsum == 0.0, 1.0, dsum), 0.5)
        interp = (1.0 - lam) * c1 + lam * c2
        f = jnp.where(any_m, mval, interp)
        tid = base + j * 16 + lane
        tmask = (tid % S) == 0
        f = jnp.where(tmask, 0.0, f)
        g = jnp.where(tmask, 0.0, 1.0 - f)
        idx = (j * 16 + lane) * 2
        plsc.store_scatter(w_v, [idx], f)
        plsc.store_scatter(w_v, [idx + 1], g)
    pltpu.sync_copy(w_v, w_hbm.at[pl.ds(base * 2, TPW * 2)])


def kernel(x, question_mask):
    B, S, D = x.shape
    BS = 512
    NBLK = S // BS
    image_tokens = x[:, 0:1, :]
    ipb = jax.nn.softmax(image_tokens, axis=-1)
    ient = -(ipb * jnp.log(ipb + 1e-08)).sum(axis=-1)
    x1 = ient / jnp.clip(ient.max(), 1e-06, None)

    ent_body = functools.partial(_ent_body, B=B, S=S, D=D, BS=BS, NBLK=NBLK)
    ent, params = pl.pallas_call(
        ent_body,
        grid=(B, NBLK),
        in_specs=[
            pl.BlockSpec((1, BS, D), lambda b, j: (b, j, 0)),
            pl.BlockSpec((B, 1), lambda b, j: (0, 0)),
        ],
        out_specs=[
            pl.BlockSpec((B, S), lambda b, j: (0, 0)),
            pl.BlockSpec((B, 48), lambda b, j: (0, 0)),
        ],
        out_shape=[
            jax.ShapeDtypeStruct((B, S), jnp.float32),
            jax.ShapeDtypeStruct((B, 48), jnp.float32),
        ],
        scratch_shapes=[
            pltpu.VMEM((B, S), jnp.float32),
            pltpu.VMEM((B, D), jnp.float32),
            pltpu.VMEM((B, D), jnp.float32),
        ],
    )(x, x1)

    NW = 32
    TPW = (B * S) // NW
    WPB = NW // B

    mesh = plsc.VectorSubcoreMesh(core_axis_name="c", subcore_axis_name="s")
    route = functools.partial(
        pl.kernel,
        out_type=jax.ShapeDtypeStruct((B * S * 2,), jnp.float32),
        mesh=mesh,
        compiler_params=pltpu.CompilerParams(needs_layout_passes=False),
        scratch_types=[
            pltpu.VMEM((TPW,), jnp.float32),
            pltpu.VMEM((48,), jnp.float32),
            pltpu.VMEM((TPW * 2,), jnp.float32),
        ],
    )(functools.partial(_route_body, TPW=TPW, S=S, WPB=WPB))
    w = route(ent.reshape(B * S), params)
    return w.reshape(B, S, 2).astype(x.dtype)


# speedup vs baseline: 1.1351x; 1.1351x over previous
import functools
import numpy as np
import jax
import jax.numpy as jnp
from jax import lax
from jax.experimental import pallas as pl
from jax.experimental.pallas import tpu as pltpu
from jax.experimental.pallas import tpu_sc as plsc

_FR = [
    ((0.0, 0.33), (0.0, 0.33), (0.0, 0.33), 0.0),
    ((0.0, 0.33), (0.0, 0.33), (0.67, 1.0), 0.333),
    ((0.0, 0.33), (0.33, 0.67), (0.33, 0.67), 0.333),
    ((0.0, 0.33), (0.67, 1.0), (0.0, 0.33), 0.333),
    ((0.0, 0.33), (0.67, 1.0), (0.67, 1.0), 0.667),
    ((0.33, 0.67), (0.0, 0.33), (0.0, 0.33), 0.333),
    ((0.33, 0.67), (0.0, 0.33), (0.67, 1.0), 0.667),
    ((0.33, 0.67), (0.33, 0.67), (0.33, 0.67), 0.5),
    ((0.33, 0.67), (0.67, 1.0), (0.0, 0.33), 0.667),
    ((0.33, 0.67), (0.67, 1.0), (0.67, 1.0), 1.0),
    ((0.67, 1.0), (0.0, 0.33), (0.0, 0.33), 0.667),
    ((0.67, 1.0), (0.0, 0.33), (0.67, 1.0), 1.0),
    ((0.67, 1.0), (0.33, 0.67), (0.33, 0.67), 1.0),
    ((0.67, 1.0), (0.67, 1.0), (0.0, 0.33), 1.167),
    ((0.67, 1.0), (0.67, 1.0), (0.67, 1.0), 1.5),
]
_LO = np.array([[r[0][0], r[1][0], r[2][0]] for r in _FR], dtype=np.float32)
_HI = np.array([[r[0][1], r[1][1], r[2][1]] for r in _FR], dtype=np.float32)
_CONS = np.array([r[3] for r in _FR], dtype=np.float32)
_CEN = (_LO + _HI) / 2.0


def _ent_body(x_ref, x1_ref, out_ent_ref, par_ref, ent_s, sum_s, img_s,
              *, B, S, D, BS, NBLK):
    b = pl.program_id(0)
    j = pl.program_id(1)

    xb = x_ref[0]
    m = jnp.max(xb, axis=1, keepdims=True)
    t = xb - m
    e = jnp.exp(t)
    Z = jnp.sum(e, axis=1, keepdims=True)
    sxe = jnp.sum(e * t, axis=1, keepdims=True)
    ent = jnp.log(Z) - sxe / Z

    ent_s[pl.ds(b, 1), pl.ds(j * BS, BS)] = ent.reshape(1, BS)

    part = jnp.sum(xb, axis=0, keepdims=True)

    @pl.when(j == 0)
    def _():
        sum_s[pl.ds(b, 1), :] = part
        img_s[pl.ds(b, 1), :] = xb[0:1, :]

    @pl.when(j != 0)
    def _():
        sum_s[pl.ds(b, 1), :] = sum_s[pl.ds(b, 1), :] + part

    @pl.when((b == B - 1) & (j == NBLK - 1))
    def _():
        ent_all = ent_s[:, :]
        img = img_s[:, :]
        tot = sum_s[:, :]
        out_ent_ref[:, :] = ent_all

        cols = jax.lax.broadcasted_iota(jnp.int32, (B, S), 1)
        ent_txt = jnp.where(cols >= 1, ent_all, -jnp.inf)
        rtx = 1.0 / jnp.maximum(jnp.max(ent_txt), 1e-6)

        bmean = (tot - img) / float(S - 1)
        dot = jnp.sum(img * bmean, axis=1, keepdims=True)
        na = jnp.sqrt(jnp.sum(img * img, axis=1, keepdims=True))
        nb = jnp.sqrt(jnp.sum(bmean * bmean, axis=1, keepdims=True))
        x3 = dot / jnp.clip(na * nb, 1e-8, None)

        lanes = jax.lax.broadcasted_iota(jnp.int32, (B, 48), 1)
        x1 = x1_ref[:, :]
        par_ref[:, :] = jnp.where(lanes < 16, x1,
                                  jnp.where(lanes < 32, x3, rtx))


def _nsqrt(d):
    bits = plsc.bitcast(d, jnp.int32)
    s = plsc.bitcast((bits >> 1) + 0x1FBD1DF5, jnp.float32)
    for _ in range(3):
        s = 0.5 * (s + d / jnp.maximum(s, 1e-30))
    return jnp.where(d > 0.0, s, 0.0)


def _route_body(ent_hbm, params_hbm, f_hbm, g_hbm, ent_v, par_v, f_v, g_v,
                *, TPW, S, WPB):
    c = lax.axis_index("c")
    s = lax.axis_index("s")
    wid = c * 16 + s
    base = wid * TPW
    pltpu.sync_copy(ent_hbm.at[pl.ds(base, TPW)], ent_v)
    pltpu.sync_copy(params_hbm.at[wid // WPB], par_v)
    x1 = par_v[pl.ds(0, 16)]
    x3 = par_v[pl.ds(16, 16)]
    rtx = par_v[pl.ds(32, 16)]
    lane = lax.iota(jnp.int32, 16)

    bb = []
    kr = []
    for r in range(15):
        lo0, _, lo2 = (float(v) for v in _LO[r])
        hi0, _, hi2 = (float(v) for v in _HI[r])
        ce0, _, ce2 = (float(v) for v in _CEN[r])
        bb.append((x1 >= lo0) & (x1 < hi0) & (x3 >= lo2) & (x3 < hi2))
        dx = x1 - ce0
        dz = x3 - ce2
        kr.append(dx * dx + dz * dz)

    for j in range(TPW // 16):
        e = ent_v[pl.ds(j * 16, 16)]
        x2 = e * rtx
        any_m = jnp.zeros((16,), jnp.bool_)
        mval = jnp.zeros((16,), jnp.float32)
        d1 = jnp.full((16,), jnp.inf, jnp.float32)
        d2 = jnp.full((16,), jnp.inf, jnp.float32)
        c1 = jnp.zeros((16,), jnp.float32)
        c2 = jnp.zeros((16,), jnp.float32)
        for r in range(15):
            lo1 = float(_LO[r][1])
            hi1 = float(_HI[r][1])
            ce1 = float(_CEN[r][1])
            cons = float(_CONS[r])
            m_r = bb[r] & (x2 >= lo1) & (x2 < hi1)
            any_m = any_m | m_r
            mval = mval + jnp.where(m_r, cons, 0.0)
            dy = x2 - ce1
            dq = dy * dy + kr[r]
            lt1 = dq < d1
            lt2 = (dq < d2) & (~lt1)
            d2n = jnp.where(lt1, d1, jnp.where(lt2, dq, d2))
            c2n = jnp.where(lt1, c1, jnp.where(lt2, cons, c2))
            d1 = jnp.where(lt1, dq, d1)
            c1 = jnp.where(lt1, cons, c1)
            d2 = d2n
            c2 = c2n
        s1 = _nsqrt(d1)
        s2 = _nsqrt(d2)
        dsum = s1 + s2
        lam = jnp.where(dsum != 0.0,
                        s1 / jnp.where(dsum == 0.0, 1.0, dsum), 0.5)
        interp = (1.0 - lam) * c1 + lam * c2
        f = jnp.where(any_m, mval, interp)
        tid = base + j * 16 + lane
        tmask = (tid % S) == 0
        f = jnp.where(tmask, 0.0, f)
        g = jnp.where(tmask, 0.0, 1.0 - f)
        f_v[pl.ds(j * 16, 16)] = f
        g_v[pl.ds(j * 16, 16)] = g
    pltpu.sync_copy(f_v, f_hbm.at[pl.ds(base, TPW)])
    pltpu.sync_copy(g_v, g_hbm.at[pl.ds(base, TPW)])


def kernel(x, question_mask):
    B, S, D = x.shape
    BS = 512
    NBLK = S // BS
    image_tokens = x[:, 0:1, :]
    ipb = jax.nn.softmax(image_tokens, axis=-1)
    ient = -(ipb * jnp.log(ipb + 1e-08)).sum(axis=-1)
    x1 = ient / jnp.clip(ient.max(), 1e-06, None)

    ent_body = functools.partial(_ent_body, B=B, S=S, D=D, BS=BS, NBLK=NBLK)
    ent, params = pl.pallas_call(
        ent_body,
        grid=(B, NBLK),
        in_specs=[
            pl.BlockSpec((1, BS, D), lambda b, j: (b, j, 0)),
            pl.BlockSpec((B, 1), lambda b, j: (0, 0)),
        ],
        out_specs=[
            pl.BlockSpec((B, S), lambda b, j: (0, 0)),
            pl.BlockSpec((B, 48), lambda b, j: (0, 0)),
        ],
        out_shape=[
            jax.ShapeDtypeStruct((B, S), jnp.float32),
            jax.ShapeDtypeStruct((B, 48), jnp.float32),
        ],
        scratch_shapes=[
            pltpu.VMEM((B, S), jnp.float32),
            pltpu.VMEM((B, D), jnp.float32),
            pltpu.VMEM((B, D), jnp.float32),
        ],
    )(x, x1)

    NW = 32
    TPW = (B * S) // NW
    WPB = NW // B

    mesh = plsc.VectorSubcoreMesh(core_axis_name="c", subcore_axis_name="s")
    route = functools.partial(
        pl.kernel,
        out_type=(
            jax.ShapeDtypeStruct((B * S,), jnp.float32),
            jax.ShapeDtypeStruct((B * S,), jnp.float32),
        ),
        mesh=mesh,
        compiler_params=pltpu.CompilerParams(needs_layout_passes=False),
        scratch_types=[
            pltpu.VMEM((TPW,), jnp.float32),
            pltpu.VMEM((48,), jnp.float32),
            pltpu.VMEM((TPW,), jnp.float32),
            pltpu.VMEM((TPW,), jnp.float32),
        ],
    )(functools.partial(_route_body, TPW=TPW, S=S, WPB=WPB))
    f, g = route(ent.reshape(B * S), params)
    return jnp.stack([f.reshape(B, S), g.reshape(B, S)], axis=-1).astype(x.dtype)
